# baseline (speedup 1.0000x reference)
import functools

import jax
import jax.numpy as jnp
from jax import lax
from jax.experimental import pallas as pl
from jax.experimental.pallas import tpu as pltpu
from jax.experimental.pallas import tpu_sc as plsc

NUM_EMB = 1000000
DIM = 32
ROWS, COLS = 16384, 50
PAD_COLS, PAD_DIM = 56, 128
B = ROWS * COLS
NC, NS = 2, 16
NW = NC * NS
R_PER_W = ROWS // NW
T_PER_C = 32
CHUNK = T_PER_C * COLS
N_CHUNKS = R_PER_W // T_PER_C


@functools.partial(
    pl.kernel,
    mesh=plsc.VectorSubcoreMesh(core_axis_name="c", subcore_axis_name="s"),
    out_type=jax.ShapeDtypeStruct((ROWS, PAD_COLS, PAD_DIM), jnp.bfloat16),
    compiler_params=pltpu.CompilerParams(use_tc_tiling_on_sc=False),
    scratch_types=[
        pltpu.VMEM((N_CHUNKS, CHUNK), jnp.int32),
        pltpu.VMEM((CHUNK, DIM), jnp.bfloat16),
        pltpu.VMEM((CHUNK, DIM), jnp.bfloat16),
        pltpu.SemaphoreType.DMA,
        pltpu.SemaphoreType.DMA,
        pltpu.SemaphoreType.DMA,
    ],
)
def _gather_sc(table_hbm, idx_hbm, out_hbm, idx_v, rows0, rows1, sem_g, sem_w0, sem_w1):
    wid = lax.axis_index("s") * NC + lax.axis_index("c")
    base = wid * R_PER_W * COLS
    for c in range(N_CHUNKS):
        pltpu.sync_copy(idx_hbm.at[pl.ds(base + c * CHUNK, CHUNK)], idx_v.at[c])

    bufs = (rows0, rows1)
    wsems = (sem_w0, sem_w1)

    def gather(c):
        return pltpu.async_copy(table_hbm.at[idx_v.at[c]], bufs[c % 2], sem_g)

    def issue_writes(c):
        t0 = wid * R_PER_W + c * T_PER_C
        buf = bufs[c % 2]
        return [
            pltpu.async_copy(
                buf.at[pl.ds(j * COLS, COLS), :],
                out_hbm.at[t0 + j, pl.ds(0, COLS), pl.ds(0, DIM)],
                wsems[c % 2],
            )
            for j in range(T_PER_C)
        ]

    writes = [None] * N_CHUNKS
    pending = gather(0)
    for c in range(N_CHUNKS):
        pending.wait()
        if c >= 1:
            for w in writes[c - 1]:
                w.wait()
        if c + 1 < N_CHUNKS:
            pending = gather(c + 1)
        writes[c] = issue_writes(c)
    for w in writes[N_CHUNKS - 1]:
        w.wait()


def kernel(token_ids, weight):
    idx = jnp.minimum(token_ids.reshape(-1), NUM_EMB - 1).astype(jnp.int32)
    out = _gather_sc(weight.astype(jnp.bfloat16), idx)
    return out[:, :COLS, :DIM].astype(jnp.float32)


# speedup vs baseline: 2.6903x; 2.6903x over previous
import functools

import jax
import jax.numpy as jnp
from jax import lax
from jax.experimental import pallas as pl
from jax.experimental.pallas import tpu as pltpu
from jax.experimental.pallas import tpu_sc as plsc

NUM_EMB = 1000000
DIM = 32
ROWS, COLS = 16384, 50
PAD_COLS, PAD_DIM = 56, 128
B = ROWS * COLS
NC, NS = 2, 16
NW = NC * NS
R_PER_W = ROWS // NW
T_PER_C = 32
CHUNK = T_PER_C * COLS
N_CHUNKS = R_PER_W // T_PER_C


@functools.partial(
    pl.kernel,
    mesh=plsc.VectorSubcoreMesh(core_axis_name="c", subcore_axis_name="s"),
    out_type=jax.ShapeDtypeStruct((ROWS, PAD_COLS, PAD_DIM), jnp.float32),
    compiler_params=pltpu.CompilerParams(use_tc_tiling_on_sc=False),
    scratch_types=[
        pltpu.VMEM((N_CHUNKS, CHUNK), jnp.int32),
        pltpu.VMEM((CHUNK, DIM), jnp.float32),
        pltpu.VMEM((CHUNK, DIM), jnp.float32),
        pltpu.SemaphoreType.DMA,
        pltpu.SemaphoreType.DMA,
        pltpu.SemaphoreType.DMA,
    ],
)
def _gather_sc(table_hbm, idx_hbm, out_hbm, idx_v, rows0, rows1, sem_g, sem_w0, sem_w1):
    wid = lax.axis_index("s") * NC + lax.axis_index("c")
    base = wid * R_PER_W * COLS
    for c in range(N_CHUNKS):
        pltpu.sync_copy(idx_hbm.at[pl.ds(base + c * CHUNK, CHUNK)], idx_v.at[c])

    bufs = (rows0, rows1)
    wsems = (sem_w0, sem_w1)

    def gather(c):
        return pltpu.async_copy(table_hbm.at[idx_v.at[c]], bufs[c % 2], sem_g)

    def issue_writes(c):
        t0 = wid * R_PER_W + c * T_PER_C
        buf = bufs[c % 2]
        return [
            pltpu.async_copy(
                buf.at[pl.ds(j * COLS, COLS), :],
                out_hbm.at[t0 + j, pl.ds(0, COLS), pl.ds(0, DIM)],
                wsems[c % 2],
            )
            for j in range(T_PER_C)
        ]

    writes = [None] * N_CHUNKS
    pending = gather(0)
    for c in range(N_CHUNKS):
        pending.wait()
        if c >= 1:
            for w in writes[c - 1]:
                w.wait()
        if c + 1 < N_CHUNKS:
            pending = gather(c + 1)
        writes[c] = issue_writes(c)
    for w in writes[N_CHUNKS - 1]:
        w.wait()


def kernel(token_ids, weight):
    idx = jnp.minimum(token_ids.reshape(-1), NUM_EMB - 1).astype(jnp.int32)
    out = _gather_sc(weight, idx)
    return out[:, :COLS, :DIM]
